# Initial kernel scaffold; baseline (speedup 1.0000x reference)
import jax
import jax.numpy as jnp
from jax.experimental import pallas as pl


def kernel(predictions):
    raise NotImplementedError("write your pallas kernel here")


# speedup vs baseline: 194.8793x; 194.8793x over previous
import jax
import jax.numpy as jnp
from jax import lax
from jax.experimental import pallas as pl
from jax.experimental.pallas import tpu as pltpu

N = 5000
NUM_CLASSES = 80
CONF_T = 0.5
NMS_T = 0.3
SIZE = 416.0
OFF = 4096.0
B = 128
NB = 40
NP = NB * B


def _decode_body(pred_ref, out_ref):
    p = pred_ref[...]
    cx = p[:, 0:1]
    cy = p[:, 1:2]
    w = p[:, 2:3]
    h = p[:, 3:4]
    obj = p[:, 4:5]
    cls = p[:, 5:]
    x1 = (cx - w * 0.5) * SIZE
    y1 = (cy - h * 0.5) * SIZE
    x2 = (cx + w * 0.5) * SIZE
    y2 = (cy + h * 0.5) * SIZE
    maxv = jnp.max(cls, axis=1, keepdims=True)
    ids = lax.broadcasted_iota(jnp.int32, cls.shape, 1)
    cid = jnp.min(jnp.where(cls == maxv, ids, NUM_CLASSES), axis=1, keepdims=True)
    offs = cid.astype(jnp.float32) * OFF
    conf = obj * maxv
    x1o = x1 + offs
    y1o = y1 + offs
    x2o = x2 + offs
    y2o = y2 + offs
    area = jnp.maximum(x2o - x1o, 0.0) * jnp.maximum(y2o - y1o, 0.0)
    out_ref[:, 0:1] = x1o
    out_ref[:, 1:2] = y1o
    out_ref[:, 2:3] = x2o
    out_ref[:, 3:4] = y2o
    out_ref[:, 4:5] = x1
    out_ref[:, 5:6] = y1
    out_ref[:, 6:7] = x2
    out_ref[:, 7:8] = y2
    out_ref[:, 8:9] = conf
    out_ref[:, 9:10] = area
    out_ref[:, 10:] = jnp.zeros_like(p[:, 10:16])


def _nms_body(cols_ref, rows_ref, keep_ref, sup_ref):
    sup_ref[...] = jnp.zeros((NB, B), jnp.float32)
    iota_s = lax.broadcasted_iota(jnp.int32, (B, B), 0)
    iota_l = lax.broadcasted_iota(jnp.int32, (B, B), 1)
    lt = iota_s < iota_l
    eq = iota_s == iota_l

    def transpose_row(v):
        return jnp.sum(jnp.where(eq, v, 0.0), axis=1, keepdims=True)

    def iou_mask(ca, cb):
        x1a, y1a, x2a, y2a, aa = ca
        x1b, y1b, x2b, y2b, ab = cb
        xx1 = jnp.maximum(x1a, x1b)
        yy1 = jnp.maximum(y1a, y1b)
        xx2 = jnp.minimum(x2a, x2b)
        yy2 = jnp.minimum(y2a, y2b)
        inter = jnp.maximum(xx2 - xx1, 0.0) * jnp.maximum(yy2 - yy1, 0.0)
        union = aa + ab - inter
        iou = inter / jnp.maximum(union, 1e-9)
        return iou > NMS_T

    def load_cols(g):
        base = g * B
        return tuple(cols_ref[pl.ds(base, B), c : c + 1] for c in range(5))

    def load_rows(h):
        return tuple(rows_ref[c, pl.ds(h, 1), :] for c in range(5))

    def g_body(g, carry):
        cg = load_cols(g)
        rg = load_rows(g)
        conf_g = rows_ref[5, pl.ds(g, 1), :]
        validg = conf_g > CONF_T
        supg = sup_ref[pl.ds(g, 1), :] > 0.0
        cand = jnp.logical_and(validg, jnp.logical_not(supg))
        candf = cand.astype(jnp.float32)
        o_gg = iou_mask(cg, rg)

        def w_cond(c):
            return c[1]

        def w_body(c):
            k, _ = c
            kcol = transpose_row(k) > 0.0
            supm = jnp.any(jnp.logical_and(jnp.logical_and(o_gg, lt), kcol),
                           axis=0, keepdims=True)
            knew = jnp.where(supm, 0.0, candf)
            return knew, jnp.any(knew != k)

        kfin, _ = lax.while_loop(w_cond, w_body, (candf, jnp.bool_(True)))
        keep_ref[pl.ds(g, 1), :] = kfin
        kcol = transpose_row(kfin) > 0.0

        def h_body(h, carry2):
            rh = load_rows(h)
            o_gh = iou_mask(cg, rh)
            supm = jnp.any(jnp.logical_and(o_gh, kcol), axis=0,
                           keepdims=True).astype(jnp.float32)
            sup_ref[pl.ds(h, 1), :] = jnp.maximum(sup_ref[pl.ds(h, 1), :], supm)
            return carry2

        lax.fori_loop(g + 1, NB, h_body, 0)
        return carry

    lax.fori_loop(0, NB, g_body, 0)


def kernel(predictions):
    pred = predictions[0]
    s1 = pl.pallas_call(
        _decode_body,
        out_shape=jax.ShapeDtypeStruct((N, 16), jnp.float32),
    )(pred)
    conf = s1[:, 8]
    order = jnp.argsort(-conf)
    ss = s1[order]
    ssp = jnp.pad(ss, ((0, NP - N), (0, 0)))
    cols = jnp.concatenate(
        [ssp[:, 0:4], ssp[:, 9:10], ssp[:, 8:9], jnp.zeros((NP, 2), jnp.float32)],
        axis=1,
    )
    rows = cols.T.reshape(8, NB, B)
    keep = pl.pallas_call(
        _nms_body,
        out_shape=jax.ShapeDtypeStruct((NB, B), jnp.float32),
        scratch_shapes=[pltpu.VMEM((NB, B), jnp.float32)],
    )(cols, rows)
    keepv = keep.reshape(NP)[:N]
    out = jnp.concatenate([ss[:, 4:8], ss[:, 8:9]], axis=1) * keepv[:, None]
    return out
